# baseline (speedup 1.0000x reference)
import jax
import jax.numpy as jnp
from jax.experimental import pallas as pl
from jax.experimental.pallas import tpu as pltpu

_PAD = 6
_K = 25
_INV_K = 1.0 / _K


def _mavg_kernel(x_ref, o_ref):
    x = x_ref[...]
    bb, L, C = x.shape
    M = o_ref.shape[1]
    z = jnp.zeros((bb, _PAD, C), x.dtype)
    xq = jnp.concatenate([z, x, z], axis=1)
    s5 = (xq[:, 0:L + 8] + xq[:, 1:L + 9] + xq[:, 2:L + 10]
          + xq[:, 3:L + 11] + xq[:, 4:L + 12])
    w = (s5[:, 1:M + 1] + s5[:, 6:M + 6] + s5[:, 11:M + 11]
         + s5[:, 16:M + 16] + s5[:, 21:M + 21])
    o_ref[...] = w * jnp.float32(_INV_K)


def kernel(x):
    B, L, C = x.shape
    L_pool = (L + 2 * _PAD - _K) // 1 + 1
    M = L_pool - 1

    bb = 16
    while B % bb:
        bb //= 2
    grid = (B // bb,)

    return pl.pallas_call(
        _mavg_kernel,
        out_shape=jax.ShapeDtypeStruct((B, M, C), x.dtype),
        grid=grid,
        in_specs=[pl.BlockSpec((bb, L, C), lambda i: (i, 0, 0))],
        out_specs=pl.BlockSpec((bb, M, C), lambda i: (i, 0, 0)),
        compiler_params=pltpu.CompilerParams(
            dimension_semantics=("parallel",),
            vmem_limit_bytes=64 * 1024 * 1024),
    )(x)


# speedup vs baseline: 1.2286x; 1.2286x over previous
import jax
import jax.numpy as jnp
from jax.experimental import pallas as pl
from jax.experimental.pallas import tpu as pltpu

_PAD = 6
_K = 25
_INV_K = 1.0 / _K


def _mavg_kernel(x_ref, o_ref):
    x = x_ref[...]
    bb, L, C = x.shape
    M = o_ref.shape[1]
    z = jnp.zeros((bb, 8, C), x.dtype)
    xp = jnp.concatenate([z, x, z], axis=1)
    s8 = xp[:, 0:L] + xp[:, 8:L + 8] + xp[:, 16:L + 16]
    p2 = s8[:, 0:L - 1] + s8[:, 1:L]
    p4 = p2[:, 0:L - 3] + p2[:, 2:L - 1]
    p8 = p4[:, 0:L - 7] + p4[:, 4:L - 3]
    w = p8[:, 3:M + 3] + xp[:, 27:M + 27]
    o_ref[...] = w * jnp.float32(_INV_K)


def kernel(x):
    B, L, C = x.shape
    L_pool = (L + 2 * _PAD - _K) // 1 + 1
    M = L_pool - 1

    bb = 8
    while B % bb:
        bb //= 2
    grid = (B // bb,)

    return pl.pallas_call(
        _mavg_kernel,
        out_shape=jax.ShapeDtypeStruct((B, M, C), x.dtype),
        grid=grid,
        in_specs=[pl.BlockSpec((bb, L, C), lambda i: (i, 0, 0))],
        out_specs=pl.BlockSpec((bb, M, C), lambda i: (i, 0, 0)),
        compiler_params=pltpu.CompilerParams(
            dimension_semantics=("parallel",),
            vmem_limit_bytes=64 * 1024 * 1024),
    )(x)
